# Initial kernel scaffold; baseline (speedup 1.0000x reference)
import jax
import jax.numpy as jnp
from jax.experimental import pallas as pl


def kernel(raw_features, node_type, W1, b1, W2, b2):
    raise NotImplementedError("write your pallas kernel here")


# speedup vs baseline: 1.2036x; 1.2036x over previous
import jax
import jax.numpy as jnp
from jax.experimental import pallas as pl
from jax.experimental.pallas import tpu as pltpu

_NUM_TYPES = 4
_BLK = 400


def _mlp_block_kernel(nt_ref, x_ref, W1_ref, b1_ref, W2_ref, b2_ref, out_ref):
    x = x_ref[...]
    nt = nt_ref[...]
    acc = jnp.zeros_like(out_ref)
    for t in range(_NUM_TYPES):
        h = jnp.maximum(
            jnp.dot(x, W1_ref[t], preferred_element_type=jnp.float32) + b1_ref[t],
            0.0,
        )
        o = jnp.dot(h, W2_ref[t], preferred_element_type=jnp.float32) + b2_ref[t]
        acc = acc + jnp.where(nt == t, o, 0.0)
    out_ref[...] = acc


def kernel(raw_features, node_type, W1, b1, W2, b2):
    n, d_in = raw_features.shape
    d_out = W2.shape[2]
    grid = n // _BLK
    nt3 = node_type.reshape(n, 1)
    return pl.pallas_call(
        _mlp_block_kernel,
        grid=(grid,),
        in_specs=[
            pl.BlockSpec((_BLK, 1), lambda i: (i, 0)),
            pl.BlockSpec((_BLK, d_in), lambda i: (i, 0)),
            pl.BlockSpec(W1.shape, lambda i: (0, 0, 0)),
            pl.BlockSpec(b1.shape, lambda i: (0, 0)),
            pl.BlockSpec(W2.shape, lambda i: (0, 0, 0)),
            pl.BlockSpec(b2.shape, lambda i: (0, 0)),
        ],
        out_specs=pl.BlockSpec((_BLK, d_out), lambda i: (i, 0)),
        out_shape=jax.ShapeDtypeStruct((n, d_out), jnp.float32),
    )(nt3, raw_features, W1, b1, W2, b2)
